# baseline (speedup 1.0000x reference)
import functools

import jax
import jax.numpy as jnp
from jax import lax
from jax.experimental import pallas as pl
from jax.experimental.pallas import tpu as pltpu
from jax.experimental.pallas import tpu_sc as plsc

NUM_E = 1000000
DIM = 32
WORDS = DIM // 4
BATCH = 16384

_info = plsc.get_sparse_core_info()
NC, NS, LANES = _info.num_cores, _info.num_subcores, _info.num_lanes
NW = NC * NS
BPW = BATCH // NW
CHUNK = 128
NCHUNK = BPW // CHUNK


def _body(qw_hbm, idx_hbm, s_hbm, zp_hbm, out_hbm,
          idx_v, widx_v, rows_w, s_v, zp_v, out_v, sem):
    c = lax.axis_index("c")
    s = lax.axis_index("s")
    wid = s * NC + c
    base = wid * BPW

    for j in range(NCHUNK):
        pltpu.sync_copy(idx_hbm.at[pl.ds(base + j * CHUNK, CHUNK)],
                        idx_v.at[j])

    for j in range(NCHUNK):
        for t in range(CHUNK // LANES):
            iv8 = idx_v[j, pl.ds(t * LANES, LANES)] * WORDS
            for w in range(WORDS):
                widx_v[j * WORDS + w, pl.ds(t * LANES, LANES)] = iv8 + w

    copies = []
    for j in range(NCHUNK):
        for w in range(WORDS):
            copies.append(pltpu.async_copy(
                qw_hbm.at[widx_v.at[j * WORDS + w]],
                rows_w.at[w, pl.ds(j * CHUNK, CHUNK)], sem))
        copies.append(pltpu.async_copy(
            s_hbm.at[idx_v.at[j]], s_v.at[pl.ds(j * CHUNK, CHUNK)], sem))
        copies.append(pltpu.async_copy(
            zp_hbm.at[idx_v.at[j]], zp_v.at[pl.ds(j * CHUNK, CHUNK)], sem))
    for cp in copies:
        cp.wait()

    iot = lax.iota(jnp.int32, LANES)

    def group(g, carry):
        r0 = g * LANES
        rvi = r0 + iot
        sv = s_v[pl.ds(r0, LANES)]
        zv = zp_v[pl.ds(r0, LANES)]
        for w in range(WORDS):
            wd = rows_w[w, pl.ds(r0, LANES)]
            for b in range(4):
                if b == 0:
                    byte = wd & 255
                elif b == 3:
                    byte = lax.shift_right_logical(wd, 24)
                else:
                    byte = lax.shift_right_logical(wd, 8 * b) & 255
                f = (byte.astype(jnp.float32) - zv) * sv
                plsc.store_scatter(
                    out_v,
                    [rvi, jnp.full((LANES,), 4 * w + b, jnp.int32)], f)
        return carry

    lax.fori_loop(0, BPW // LANES, group, 0)

    pltpu.sync_copy(out_v, out_hbm.at[pl.ds(base, BPW)])


def _run(qw1d, idx, scales, zps):
    mesh = plsc.VectorSubcoreMesh(core_axis_name="c", subcore_axis_name="s")
    k = functools.partial(
        pl.kernel,
        out_type=jax.ShapeDtypeStruct((BATCH, DIM), jnp.float32),
        mesh=mesh,
        scratch_types=[
            pltpu.VMEM((NCHUNK, CHUNK), jnp.int32),
            pltpu.VMEM((NCHUNK * WORDS, CHUNK), jnp.int32),
            pltpu.VMEM((WORDS, BPW), jnp.int32),
            pltpu.VMEM((BPW,), jnp.float32),
            pltpu.VMEM((BPW,), jnp.float32),
            pltpu.VMEM((BPW, DIM), jnp.float32),
            pltpu.SemaphoreType.DMA,
        ],
        compiler_params=pltpu.CompilerParams(
            needs_layout_passes=False, use_tc_tiling_on_sc=False),
    )(_body)
    return k(qw1d, idx, scales, zps)


def kernel(indices, qweight, scales, zero_points):
    qw1d = lax.bitcast_convert_type(
        qweight.reshape(NUM_E * WORDS, 4), jnp.int32)
    idx = indices.astype(jnp.int32)
    return _run(qw1d, idx, scales, zero_points)


# speedup vs baseline: 7.5906x; 7.5906x over previous
import functools

import jax
import jax.numpy as jnp
from jax import lax
from jax.experimental import pallas as pl
from jax.experimental.pallas import tpu as pltpu
from jax.experimental.pallas import tpu_sc as plsc

NUM_E = 1000000
DIM = 32
WORDS = DIM // 4
BATCH = 16384

_info = plsc.get_sparse_core_info()
NC, NS, LANES = _info.num_cores, _info.num_subcores, _info.num_lanes
NW = NC * NS
BPW = BATCH // NW
CHUNK = 128
NCHUNK = BPW // CHUNK
WAVE = 64


def _body(qw_hbm, idx_hbm, s_hbm, zp_hbm, out_hbm,
          idx_v, rows_u8, s_v, zp_v, out_v, sem, sem2):
    c = lax.axis_index("c")
    s = lax.axis_index("s")
    wid = s * NC + c
    base = wid * BPW

    for j in range(NCHUNK):
        pltpu.sync_copy(idx_hbm.at[pl.ds(base + j * CHUNK, CHUNK)],
                        idx_v.at[j])

    copies = []
    for j in range(NCHUNK):
        copies.append(pltpu.async_copy(
            s_hbm.at[idx_v.at[j]], s_v.at[pl.ds(j * CHUNK, CHUNK)], sem2))
        copies.append(pltpu.async_copy(
            zp_hbm.at[idx_v.at[j]], zp_v.at[pl.ds(j * CHUNK, CHUNK)], sem2))

    def drain_one():
        pltpu.make_async_copy(
            qw_hbm.at[0], rows_u8.at[pl.ds(0, DIM)], sem
        ).wait()

    def wave(gg, carry):
        ivv = idx_v[gg >> 3, pl.ds((gg & 7) * LANES, LANES)]
        for u in range(LANES):
            i = gg * LANES + u
            pltpu.async_copy(
                qw_hbm.at[ivv[u]], rows_u8.at[pl.ds(i * DIM, DIM)], sem)

        @pl.when(gg > 0)
        def _():
            for _u in range(LANES):
                drain_one()

        return carry

    lax.fori_loop(0, BPW // LANES, wave, 0)
    for _u in range(LANES):
        drain_one()
    for cp in copies:
        cp.wait()

    iot = lax.iota(jnp.int32, LANES)
    iot_hi = lax.shift_right_logical(iot, 3)
    col_base = (iot & 7) * 4

    def pair(k, carry):
        rv = 2 * k + iot_hi
        wv = plsc.bitcast(rows_u8[pl.ds(k * 2 * DIM, 2 * DIM)], jnp.int32)
        sv = plsc.load_gather(s_v, [rv])
        zv = plsc.load_gather(zp_v, [rv])
        for b in range(4):
            if b == 0:
                byte = wv & 255
            elif b == 3:
                byte = lax.shift_right_logical(wv, 24)
            else:
                byte = lax.shift_right_logical(wv, 8 * b) & 255
            f = (byte.astype(jnp.float32) - zv) * sv
            plsc.store_scatter(out_v, [rv, col_base + b], f)
        return carry

    lax.fori_loop(0, BPW // 2, pair, 0)

    pltpu.sync_copy(out_v, out_hbm.at[pl.ds(base, BPW)])


def _run(qweight, idx, scales, zps):
    mesh = plsc.VectorSubcoreMesh(core_axis_name="c", subcore_axis_name="s")
    k = functools.partial(
        pl.kernel,
        out_type=jax.ShapeDtypeStruct((BATCH, DIM), jnp.float32),
        mesh=mesh,
        scratch_types=[
            pltpu.VMEM((NCHUNK, CHUNK), jnp.int32),
            pltpu.VMEM((BPW * DIM,), jnp.uint8),
            pltpu.VMEM((BPW,), jnp.float32),
            pltpu.VMEM((BPW,), jnp.float32),
            pltpu.VMEM((BPW, DIM), jnp.float32),
            pltpu.SemaphoreType.DMA,
            pltpu.SemaphoreType.DMA,
        ],
        compiler_params=pltpu.CompilerParams(
            needs_layout_passes=False, use_tc_tiling_on_sc=False),
    )(_body)
    return k(qweight, idx, scales, zps)


def kernel(indices, qweight, scales, zero_points):
    idx = indices.astype(jnp.int32)
    return _run(qweight, idx, scales, zero_points)
